# baseline (speedup 1.0000x reference)
import functools

import jax
import jax.numpy as jnp
from jax import lax
from jax.experimental import pallas as pl
from jax.experimental.pallas import tpu as pltpu
from jax.experimental.pallas import tpu_sc as plsc

N = 320000
D = 128
S = 10000
H = S // 2
CHUNK = 128
NCH = N // CHUNK
NC = 2
NS = 16
NJUNK = 8
NRANK = 16
ACC_ROWS = H + NJUNK
IDX_T = 160
SUB_ROWS = 312
TAIL_Z = ACC_ROWS - NS * SUB_ROWS
TAIL_O = H - NS * SUB_ROWS
NV = D // 16


def _sc_segment_sum(values, ranks, tgts):
    mesh = plsc.VectorSubcoreMesh(core_axis_name="c", subcore_axis_name="s")

    @functools.partial(
        pl.kernel,
        mesh=mesh,
        out_type=jax.ShapeDtypeStruct((S, D), jnp.float32),
        scratch_types=[
            pltpu.VMEM((IDX_T, CHUNK), jnp.int32),
            pltpu.VMEM((IDX_T, NRANK), jnp.int32),
            pltpu.VMEM((2, CHUNK, D), jnp.float32),
            pltpu.VMEM((NRANK, D), jnp.float32),
            pltpu.VMEM_SHARED((ACC_ROWS, D), jnp.float32),
            pltpu.SemaphoreType.DMA,
        ],
    )
    def k(vals_hbm, rank_hbm, tgt_hbm, out_hbm,
          rank_v, tgt_v, buf_v, stage_v, acc_sh, sem_in):
        c = lax.axis_index("c")
        s = lax.axis_index("s")

        def zrow(i, carry):
            for kk in range(NV):
                buf_v[0, i, pl.ds(16 * kk, 16)] = jnp.zeros((16,),
                                                            jnp.float32)
            return carry
        lax.fori_loop(0, CHUNK, zrow, 0)
        for r, zn in enumerate((CHUNK, CHUNK, SUB_ROWS - 2 * CHUNK)):
            pltpu.sync_copy(
                buf_v.at[0, pl.ds(0, zn)],
                acc_sh.at[pl.ds(s * SUB_ROWS + r * CHUNK, zn)])

        @pl.when(s == NS - 1)
        def _():
            pltpu.sync_copy(buf_v.at[0, pl.ds(0, TAIL_Z)],
                            acc_sh.at[pl.ds(NS * SUB_ROWS, TAIL_Z)])

        pltpu.sync_copy(rank_hbm.at[c, s], rank_v)
        pltpu.sync_copy(tgt_hbm.at[c, s], tgt_v)
        plsc.subcore_barrier()

        def flag(t):
            return rank_v[t, pl.ds(0, 16)][0]

        def relevant(t):
            return flag(t) != 16

        def start(t):
            ch = t * NS + s
            pltpu.make_async_copy(
                vals_hbm.at[pl.ds(ch * CHUNK, CHUNK)],
                buf_v.at[lax.rem(t, 2)], sem_in).start()

        @pl.when(relevant(0))
        def _():
            start(0)

        def reduce_chunk(t, slot):
            zero = jnp.zeros((16,), jnp.float32)
            acc = [zero] * NV
            r_prev = jnp.int32(-1)
            for g in range(CHUNK // 16):
                rk = rank_v[t, pl.ds(g * 16, 16)]
                for j in range(16):
                    r = rk[j]
                    same = r == r_prev
                    for kk in range(NV):
                        row = buf_v[slot, g * 16 + j, pl.ds(16 * kk, 16)]
                        accv = jnp.where(same, acc[kk] + row, row)
                        stage_v[r, pl.ds(16 * kk, 16)] = accv
                        acc[kk] = accv
                    r_prev = r
            pltpu.sync_copy(stage_v, acc_sh.at[tgt_v.at[t]], add=True)

        def body(t, carry):
            tcur = jnp.minimum(t, IDX_T - 1)
            tp1 = jnp.minimum(t + 1, IDX_T - 1)

            @pl.when((t + 1 < IDX_T) & relevant(tp1))
            def _():
                start(tp1)

            @pl.when(relevant(tcur))
            def _():
                slot = lax.rem(tcur, 2)
                pltpu.make_async_copy(
                    vals_hbm.at[pl.ds(0, CHUNK)], buf_v.at[slot],
                    sem_in).wait()
                fb = flag(tcur) >= 32

                @pl.when(fb)
                def _():
                    for kk in range(CHUNK // 16):
                        v = rank_v[tcur, pl.ds(16 * kk, 16)]
                        rank_v[tcur, pl.ds(16 * kk, 16)] = v - 32
                    pltpu.sync_copy(buf_v.at[slot],
                                    acc_sh.at[rank_v.at[tcur]], add=True)

                @pl.when(jnp.logical_not(fb))
                def _():
                    reduce_chunk(tcur, slot)
            return carry

        lax.fori_loop(0, IDX_T, body, 0)

        plsc.subcore_barrier()
        pltpu.sync_copy(
            acc_sh.at[pl.ds(s * SUB_ROWS, SUB_ROWS)],
            out_hbm.at[pl.ds(c * H + s * SUB_ROWS, SUB_ROWS)])

        @pl.when(s == NS - 1)
        def _():
            pltpu.sync_copy(
                acc_sh.at[pl.ds(NS * SUB_ROWS, TAIL_O)],
                out_hbm.at[pl.ds(c * H + NS * SUB_ROWS, TAIL_O)])

    return k(values, ranks, tgts)


def _worker_layout(arr, pad_val):
    pad = jnp.full((IDX_T * NS - NCH, arr.shape[1]), pad_val, jnp.int32)
    arr = jnp.concatenate([arr.astype(jnp.int32), pad], axis=0)
    return arr.reshape(IDX_T, NS, arr.shape[1]).transpose(1, 0, 2)


def _chunk_meta(adj):
    change = jnp.concatenate(
        [jnp.ones((NCH, 1), jnp.bool_), adj[:, 1:] != adj[:, :-1]], axis=1)
    rank_raw = jnp.cumsum(change.astype(jnp.int32), axis=1) - 1
    fb = rank_raw[:, -1:] >= NRANK
    irrel = jnp.min(adj, axis=1, keepdims=True) >= H
    rank = jnp.minimum(rank_raw, NRANK - 1)
    rank = jnp.where(fb, adj + 32, rank)
    rank = jnp.where(irrel, NRANK, rank)
    rows = jnp.broadcast_to(jnp.arange(NCH)[:, None], (NCH, CHUNK))
    tgt = jnp.broadcast_to(
        H + (jnp.arange(NRANK + 1, dtype=jnp.int32) % NJUNK)[None, :],
        (NCH, NRANK + 1))
    tgt = tgt.at[rows, jnp.minimum(rank_raw, NRANK)].set(adj)[:, :NRANK]
    junk_row = H + (jnp.arange(NRANK, dtype=jnp.int32) % NJUNK)[None, :]
    tgt = jnp.where(fb | irrel, junk_row, tgt)
    return rank, tgt


def kernel(values, segment_ids):
    seg2d = segment_ids.astype(jnp.int32).reshape(NCH, CHUNK)
    junk = H + (jnp.arange(CHUNK, dtype=jnp.int32) % NJUNK)[None, :]
    adj0 = jnp.where(seg2d < H, seg2d, junk)
    adj1 = jnp.where(seg2d >= H, seg2d - H, junk)
    rank0, tgt0 = _chunk_meta(adj0)
    rank1, tgt1 = _chunk_meta(adj1)
    ranks = jnp.stack([_worker_layout(rank0, NRANK),
                       _worker_layout(rank1, NRANK)])
    tgts = jnp.stack([_worker_layout(tgt0, H), _worker_layout(tgt1, H)])
    return _sc_segment_sum(values, ranks, tgts)
